# Initial kernel scaffold; baseline (speedup 1.0000x reference)
import jax
import jax.numpy as jnp
from jax.experimental import pallas as pl


def kernel(x, edge_index, W1, att_src1, att_dst1, b1, W2, att_src2, att_dst2, b2):
    raise NotImplementedError("write your pallas kernel here")


# speedup vs baseline: 1.0710x; 1.0710x over previous
import functools

import jax
import jax.numpy as jnp
from jax.experimental import pallas as pl
from jax.experimental.pallas import tpu as pltpu

N = 10000
E = 320000
H = 8
C = 128


def _mm_body(x_ref, w_ref, o_ref):
    o_ref[...] = jnp.dot(x_ref[...], w_ref[...],
                         preferred_element_type=jnp.float32)


def _matmul(x, w, bm=400):
    m, k = x.shape
    k2, n = w.shape
    grid = (m // bm,)
    return pl.pallas_call(
        _mm_body,
        grid=grid,
        in_specs=[
            pl.BlockSpec((bm, k), lambda i: (i, 0)),
            pl.BlockSpec((k, n), lambda i: (0, 0)),
        ],
        out_specs=pl.BlockSpec((bm, n), lambda i: (i, 0)),
        out_shape=jax.ShapeDtypeStruct((m, n), jnp.float32),
    )(x, w)


def _gat_layer(x, src, dst, W, att_src, att_dst, bias, concat):
    n = x.shape[0]
    h, c = att_src.shape
    hx2d = _matmul(x, W)
    hx = hx2d.reshape(n, h, c)
    a_src = (hx * att_src[None]).sum(-1)
    a_dst = (hx * att_dst[None]).sum(-1)
    alpha = a_src[src] + a_dst[dst]
    alpha = jax.nn.leaky_relu(alpha, negative_slope=0.2)
    ex = jnp.exp(alpha)
    denom = jax.ops.segment_sum(ex, dst, num_segments=n)
    msg = hx[src] * ex[..., None]
    out = jax.ops.segment_sum(msg, dst, num_segments=n)
    out = out / (denom[..., None] + 1e-16)
    if concat:
        out = out.reshape(n, h * c)
    else:
        out = out.mean(axis=1)
    return out + bias


def kernel(x, edge_index, W1, att_src1, att_dst1, b1, W2, att_src2,
           att_dst2, b2):
    loop = jnp.arange(N, dtype=edge_index.dtype)
    src = jnp.concatenate([edge_index[0], loop])
    dst = jnp.concatenate([edge_index[1], loop])
    h = _gat_layer(x, src, dst, W1, att_src1, att_dst1, b1, concat=True)
    h = jax.nn.elu(h)
    out = _gat_layer(h, src, dst, W2, att_src2, att_dst2, b2, concat=False)
    return out


# speedup vs baseline: 7.6306x; 7.1246x over previous
import functools

import jax
import jax.numpy as jnp
from jax import lax
from jax.experimental import pallas as pl
from jax.experimental.pallas import tpu as pltpu
from jax.experimental.pallas import tpu_sc as plsc

N = 10000
E = 320000
H = 8
C = 128
DIN = 128

NPAD = 10240
DUMMY = N
NS = 16
NC = 2
CH = 20992
EP = NS * CH
EPR = EP // 128
BA = 256
KA = BA // 128
NBA = CH // BA
BB = 128
NBB = CH // BB
RPT = NPAD // NS


def _zero_fbuf(fbuf):
    @pl.loop(0, 128)
    def _(r):
        for k in range(8):
            fbuf[r, pl.ds(k * 16, 16)] = jnp.zeros((16,), jnp.float32)


def _sc_body(hx_hbm, asrc_hbm, adst_hbm, src_hbm, dst_hbm, bias_hbm,
             out_hbm, ex_hbm,
             ia_src, ia_dst, ga_src, ga_dst, ex_a,
             ib_src, ib_dst, exb, gb, dnb, bias_v,
             acc_sp, dn_sp, sem, *, elu):
    c = lax.axis_index("c")
    s = lax.axis_index("s")
    h0 = c * 4
    estart = s * CH
    rstart = s * (CH // 128)
    r0 = s * RPT

    _zero_fbuf(gb)

    @pl.loop(0, 128)
    def _(r):
        dnb[r, :] = jnp.zeros((16,), jnp.float32)

    for rc in range(RPT // 128):
        pltpu.sync_copy(gb, acc_sp.at[pl.ds(r0 + rc * 128, 128), :])
        pltpu.sync_copy(dnb, dn_sp.at[pl.ds(r0 + rc * 128, 128), :])
    plsc.subcore_barrier()

    @pl.loop(0, NBA)
    def _(ba):
        off = estart + ba * BA
        row = rstart + ba * KA
        pltpu.sync_copy(src_hbm.at[pl.ds(row, KA), :], ia_src)
        pltpu.sync_copy(dst_hbm.at[pl.ds(row, KA), :], ia_dst)
        cps = []
        for j in range(KA):
            cps.append(pltpu.async_copy(
                asrc_hbm.at[ia_src.at[j]],
                ga_src.at[pl.ds(j * 128, 128), :], sem))
            cps.append(pltpu.async_copy(
                adst_hbm.at[ia_dst.at[j]],
                ga_dst.at[pl.ds(j * 128, 128), :], sem))
        for cp in cps:
            cp.wait()

        @pl.loop(0, BA, unroll=2)
        def _(i):
            v = ga_src[i, :] + ga_dst[i, :]
            v = jnp.where(v > 0.0, v, 0.2 * v)
            ex_a[i, :] = jnp.exp(v)

        pltpu.sync_copy(ex_a, ex_hbm.at[c, pl.ds(off, BA), :])
        for j in range(KA):
            pltpu.sync_copy(ex_a.at[pl.ds(j * 128, 128), :],
                            dn_sp.at[ia_dst.at[j]], add=True)

    plsc.subcore_barrier()

    for hl in range(4):
        h = h0 + hl

        @pl.loop(0, NBB)
        def _(b):
            off = estart + b * BB
            row = rstart + b
            pltpu.sync_copy(src_hbm.at[pl.ds(row, 1), :], ib_src)
            pltpu.sync_copy(dst_hbm.at[pl.ds(row, 1), :], ib_dst)
            pltpu.sync_copy(ex_hbm.at[c, pl.ds(off, BB), :], exb)
            pltpu.async_copy(hx_hbm.at[h].at[ib_src.at[0]], gb, sem).wait()

            hvec = jnp.full((16,), h, jnp.int32)

            @pl.loop(0, BB)
            def _(e):
                w = plsc.load_gather(exb, [jnp.full((16,), e, jnp.int32),
                                           hvec])
                for k in range(8):
                    gb[e, pl.ds(k * 16, 16)] = gb[e, pl.ds(k * 16, 16)] * w

            pltpu.sync_copy(gb, acc_sp.at[ib_dst.at[0]], add=True)

        plsc.subcore_barrier()

        pltpu.sync_copy(bias_hbm.at[h], bias_v)
        hvec = jnp.full((16,), h, jnp.int32)
        for rc in range(RPT // 128):
            rr = r0 + rc * 128
            pltpu.sync_copy(acc_sp.at[pl.ds(rr, 128), :], gb)
            pltpu.sync_copy(dn_sp.at[pl.ds(rr, 128), :], dnb)

            @pl.loop(0, 128)
            def _(r):
                d = plsc.load_gather(
                    dnb, [jnp.full((16,), r, jnp.int32), hvec])
                d = d + 1e-16
                for k in range(8):
                    v = gb[r, pl.ds(k * 16, 16)] / d
                    v = v + bias_v[pl.ds(k * 16, 16)]
                    if elu:
                        v = jnp.where(
                            v > 0.0, v,
                            jnp.exp(jnp.minimum(v, 0.0)) - 1.0)
                    gb[r, pl.ds(k * 16, 16)] = v

            pltpu.sync_copy(gb, out_hbm.at[h, pl.ds(rr, 128), :])

        _zero_fbuf(gb)
        for rc in range(RPT // 128):
            pltpu.sync_copy(gb, acc_sp.at[pl.ds(r0 + rc * 128, 128), :])
        plsc.subcore_barrier()


def _sc_layer(hx, asrc16, adst16, src2d, dst2d, bias_h, elu):
    mesh = plsc.VectorSubcoreMesh(core_axis_name="c", subcore_axis_name="s",
                                  num_cores=NC, num_subcores=NS)
    f = pl.kernel(
        functools.partial(_sc_body, elu=elu),
        out_type=(
            jax.ShapeDtypeStruct((H, NPAD, C), jnp.float32),
            jax.ShapeDtypeStruct((NC, EP, 16), jnp.float32),
        ),
        mesh=mesh,
        compiler_params=pltpu.CompilerParams(needs_layout_passes=False,
                                             use_tc_tiling_on_sc=False),
        scratch_types=[
            pltpu.VMEM((KA, 128), jnp.int32),
            pltpu.VMEM((KA, 128), jnp.int32),
            pltpu.VMEM((BA, 16), jnp.float32),
            pltpu.VMEM((BA, 16), jnp.float32),
            pltpu.VMEM((BA, 16), jnp.float32),
            pltpu.VMEM((1, 128), jnp.int32),
            pltpu.VMEM((1, 128), jnp.int32),
            pltpu.VMEM((BB, 16), jnp.float32),
            pltpu.VMEM((BB, C), jnp.float32),
            pltpu.VMEM((128, 16), jnp.float32),
            pltpu.VMEM((C,), jnp.float32),
            pltpu.VMEM_SHARED((NPAD, C), jnp.float32),
            pltpu.VMEM_SHARED((NPAD, 16), jnp.float32),
            pltpu.SemaphoreType.DMA,
        ],
    )
    out, _ex = f(hx, asrc16, adst16, src2d, dst2d, bias_h)
    return out


def _mm1_body(x_ref, w_ref, o_ref):
    o_ref[0] = jnp.dot(x_ref[...], w_ref[...],
                       preferred_element_type=jnp.float32)


def _mm1(x, wcat, bm=512):
    nj = wcat.shape[1] // 128
    grid = (NPAD // bm, nj)
    return pl.pallas_call(
        _mm1_body,
        grid=grid,
        in_specs=[
            pl.BlockSpec((bm, x.shape[1]), lambda i, j: (i, 0)),
            pl.BlockSpec((x.shape[1], 128), lambda i, j: (0, j)),
        ],
        out_specs=pl.BlockSpec((1, bm, 128), lambda i, j: (j, i, 0)),
        out_shape=jax.ShapeDtypeStruct((nj, NPAD, 128), jnp.float32),
    )(x, wcat)


def _mm2_body(x_ref, w_ref, o_ref):
    k = pl.program_id(2)
    p = jnp.dot(x_ref[0], w_ref[...], preferred_element_type=jnp.float32)

    @pl.when(k == 0)
    def _():
        o_ref[0] = p

    @pl.when(k != 0)
    def _():
        o_ref[0] = o_ref[0] + p


def _mm2(x_heads, wcat, bm=512):
    nj = wcat.shape[1] // 128
    grid = (NPAD // bm, nj, H)
    return pl.pallas_call(
        _mm2_body,
        grid=grid,
        in_specs=[
            pl.BlockSpec((1, bm, 128), lambda i, j, k: (k, i, 0)),
            pl.BlockSpec((128, 128), lambda i, j, k: (k, j)),
        ],
        out_specs=pl.BlockSpec((1, bm, 128), lambda i, j, k: (j, i, 0)),
        out_shape=jax.ShapeDtypeStruct((nj, NPAD, 128), jnp.float32),
    )(x_heads, wcat)


def _ep_body(x_ref, b_ref, o_ref):
    o_ref[...] = jnp.mean(x_ref[...], axis=0) + b_ref[...]


def _epilogue(out_heads, b2, bm=400):
    return pl.pallas_call(
        _ep_body,
        grid=(N // bm,),
        in_specs=[
            pl.BlockSpec((H, bm, C), lambda i: (0, i, 0)),
            pl.BlockSpec((1, C), lambda i: (0, 0)),
        ],
        out_specs=pl.BlockSpec((bm, C), lambda i: (i, 0)),
        out_shape=jax.ShapeDtypeStruct((N, C), jnp.float32),
    )(out_heads, b2.reshape(1, C))


def _att_fold(W, att_src, att_dst):
    Wr = W.reshape(W.shape[0], H, C)
    wsrc = jnp.einsum("khc,hc->kh", Wr, att_src)
    wdst = jnp.einsum("khc,hc->kh", Wr, att_dst)
    pad = jnp.zeros((W.shape[0], 112), jnp.float32)
    return jnp.concatenate([W, wsrc, wdst, pad], axis=1)


def kernel(x, edge_index, W1, att_src1, att_dst1, b1, W2, att_src2,
           att_dst2, b2):
    loop = jnp.arange(N, dtype=edge_index.dtype)
    src = jnp.concatenate([edge_index[0], loop]).astype(jnp.int32)
    dst = jnp.concatenate([edge_index[1], loop]).astype(jnp.int32)
    npad_e = EP - src.shape[0]
    fill = jnp.full((npad_e,), DUMMY, jnp.int32)
    src2d = jnp.concatenate([src, fill]).reshape(EPR, 128)
    dst2d = jnp.concatenate([dst, fill]).reshape(EPR, 128)

    x_pad = jnp.zeros((NPAD, DIN), jnp.float32).at[:N].set(x)

    mm1 = _mm1(x_pad, _att_fold(W1, att_src1, att_dst1))
    hx1 = mm1[:H]
    a1 = mm1[H]
    asrc1 = jnp.tile(a1[:, 0:8], (1, 2))
    adst1 = jnp.tile(a1[:, 8:16], (1, 2))
    out1 = _sc_layer(hx1, asrc1, adst1, src2d, dst2d,
                     b1.reshape(H, C), elu=True)

    mm2 = _mm2(out1, _att_fold(W2, att_src2, att_dst2))
    hx2 = mm2[:H]
    a2 = mm2[H]
    asrc2 = jnp.tile(a2[:, 0:8], (1, 2))
    adst2 = jnp.tile(a2[:, 8:16], (1, 2))
    out2 = _sc_layer(hx2, asrc2, adst2, src2d, dst2d,
                     jnp.zeros((H, C), jnp.float32), elu=False)

    return _epilogue(out2, b2)


# speedup vs baseline: 8.8073x; 1.1542x over previous
import functools

import jax
import jax.numpy as jnp
from jax import lax
from jax.experimental import pallas as pl
from jax.experimental.pallas import tpu as pltpu
from jax.experimental.pallas import tpu_sc as plsc

N = 10000
E = 320000
H = 8
C = 128
DIN = 128

NPAD = 10240
DUMMY = N
NS = 16
NC = 2
CH = 20992
EP = NS * CH
EPR = EP // 128
BA = 256
KA = BA // 128
NBA = CH // BA
BB = 128
NSB = CH // BB
RPT = NPAD // NS


def _zero_fbuf(fbuf):
    @pl.loop(0, 128)
    def _(r):
        for k in range(8):
            fbuf[r, pl.ds(k * 16, 16)] = jnp.zeros((16,), jnp.float32)


def _sc_body(hx_hbm, asrc_hbm, adst_hbm, src_hbm, dst_hbm, bias_hbm,
             out_hbm, ex_hbm,
             ia_src, ia_dst, ga_src, ga_dst, ex_a,
             ibsA, ibdA, ibsB, ibdB, exbA, exbB, gb, dnb, bias_v,
             acc_sp, dn_sp, sem, s0, s1, *, elu):
    c = lax.axis_index("c")
    s = lax.axis_index("s")
    h0 = c * 4
    estart = s * CH
    rstart = s * (CH // 128)
    r0 = s * RPT

    _zero_fbuf(gb)

    @pl.loop(0, 128)
    def _(r):
        dnb[r, :] = jnp.zeros((16,), jnp.float32)

    for rc in range(RPT // 128):
        pltpu.sync_copy(gb, acc_sp.at[pl.ds(r0 + rc * 128, 128), :])
        pltpu.sync_copy(dnb, dn_sp.at[pl.ds(r0 + rc * 128, 128), :])
    plsc.subcore_barrier()

    @pl.loop(0, NBA)
    def _(ba):
        off = estart + ba * BA
        row = rstart + ba * KA
        pltpu.sync_copy(src_hbm.at[pl.ds(row, KA), :], ia_src)
        pltpu.sync_copy(dst_hbm.at[pl.ds(row, KA), :], ia_dst)
        cps = []
        for j in range(KA):
            cps.append(pltpu.async_copy(
                asrc_hbm.at[ia_src.at[j]],
                ga_src.at[pl.ds(j * 128, 128), :], sem))
            cps.append(pltpu.async_copy(
                adst_hbm.at[ia_dst.at[j]],
                ga_dst.at[pl.ds(j * 128, 128), :], sem))
        for cp in cps:
            cp.wait()

        @pl.loop(0, BA, unroll=2)
        def _(i):
            v = ga_src[i, :] + ga_dst[i, :]
            v = jnp.where(v > 0.0, v, 0.2 * v)
            ex_a[i, :] = jnp.exp(v)

        pltpu.sync_copy(ex_a, ex_hbm.at[c, pl.ds(off, BA), :])
        for j in range(KA):
            pltpu.sync_copy(ex_a.at[pl.ds(j * 128, 128), :],
                            dn_sp.at[ia_dst.at[j]], add=True)

    plsc.subcore_barrier()

    g0 = gb.at[pl.ds(0, 64), :]
    g1 = gb.at[pl.ds(64, 64), :]

    def _gissue(hxh, ib, lo, sl, sm):
        pltpu.async_copy(hxh.at[ib.at[0, pl.ds(lo, 64)]], sl, sm)

    def _gwait(hxh, ib, lo, sl, sm):
        pltpu.make_async_copy(hxh.at[ib.at[0, pl.ds(lo, 64)]], sl, sm).wait()

    def _ldidx(row, ibs, ibd, off, exb):
        pltpu.sync_copy(src_hbm.at[pl.ds(row, 1), :], ibs)
        pltpu.sync_copy(dst_hbm.at[pl.ds(row, 1), :], ibd)
        pltpu.sync_copy(ex_hbm.at[c, pl.ds(off, BB), :], exb)

    @pl.loop(0, 4)
    def _(hl):
        h = h0 + hl
        hxh = hx_hbm.at[h]
        hvec = jnp.full((16,), h, jnp.int32)

        def _scale(base, exb):
            @pl.loop(0, 64)
            def _(e):
                w = plsc.load_gather(
                    exb, [jnp.full((16,), e, jnp.int32) + base, hvec])
                for k in range(8):
                    gb[base + e, pl.ds(k * 16, 16)] = (
                        gb[base + e, pl.ds(k * 16, 16)] * w)

        def _proc(ibs, ibd, exb, hxh=hxh):
            _gwait(hxh, ibs, 0, g0, s0)
            _scale(0, exb)
            _gwait(hxh, ibs, 64, g1, s1)
            _scale(64, exb)
            pltpu.sync_copy(gb, acc_sp.at[ibd.at[0]], add=True)

        def _fire(ibs, hxh=hxh):
            _gissue(hxh, ibs, 0, g0, s0)
            _gissue(hxh, ibs, 64, g1, s1)

        _ldidx(rstart, ibsA, ibdA, estart, exbA)
        _fire(ibsA)

        @pl.loop(0, NSB // 2 - 1)
        def _(j):
            row = rstart + 2 * j
            off = estart + j * (2 * BB)
            _ldidx(row + 1, ibsB, ibdB, off + BB, exbB)
            _proc(ibsA, ibdA, exbA)
            _fire(ibsB)
            _ldidx(row + 2, ibsA, ibdA, off + 2 * BB, exbA)
            _proc(ibsB, ibdB, exbB)
            _fire(ibsA)

        _ldidx(rstart + NSB - 1, ibsB, ibdB, estart + (NSB - 1) * BB, exbB)
        _proc(ibsA, ibdA, exbA)
        _fire(ibsB)
        _proc(ibsB, ibdB, exbB)

        plsc.subcore_barrier()

        pltpu.sync_copy(bias_hbm.at[h], bias_v)
        hvec = jnp.full((16,), h, jnp.int32)
        for rc in range(RPT // 128):
            rr = r0 + rc * 128
            pltpu.sync_copy(acc_sp.at[pl.ds(rr, 128), :], gb)
            pltpu.sync_copy(dn_sp.at[pl.ds(rr, 128), :], dnb)

            @pl.loop(0, 128)
            def _(r):
                d = plsc.load_gather(
                    dnb, [jnp.full((16,), r, jnp.int32), hvec])
                d = 1.0 / (d + 1e-16)
                for k in range(8):
                    v = gb[r, pl.ds(k * 16, 16)] * d
                    v = v + bias_v[pl.ds(k * 16, 16)]
                    if elu:
                        v = jnp.where(
                            v > 0.0, v,
                            jnp.exp(jnp.minimum(v, 0.0)) - 1.0)
                    gb[r, pl.ds(k * 16, 16)] = v

            pltpu.sync_copy(gb, out_hbm.at[h, pl.ds(rr, 128), :])

        _zero_fbuf(gb)
        for rc in range(RPT // 128):
            pltpu.sync_copy(gb, acc_sp.at[pl.ds(r0 + rc * 128, 128), :])
        plsc.subcore_barrier()


def _sc_layer(hx, asrc16, adst16, src2d, dst2d, bias_h, elu):
    mesh = plsc.VectorSubcoreMesh(core_axis_name="c", subcore_axis_name="s",
                                  num_cores=NC, num_subcores=NS)
    f = pl.kernel(
        functools.partial(_sc_body, elu=elu),
        out_type=(
            jax.ShapeDtypeStruct((H, NPAD, C), jnp.float32),
            jax.ShapeDtypeStruct((NC, EP, 16), jnp.float32),
        ),
        mesh=mesh,
        compiler_params=pltpu.CompilerParams(needs_layout_passes=False,
                                             use_tc_tiling_on_sc=False),
        scratch_types=[
            pltpu.VMEM((KA, 128), jnp.int32),
            pltpu.VMEM((KA, 128), jnp.int32),
            pltpu.VMEM((BA, 16), jnp.float32),
            pltpu.VMEM((BA, 16), jnp.float32),
            pltpu.VMEM((BA, 16), jnp.float32),
            pltpu.VMEM((1, 128), jnp.int32),
            pltpu.VMEM((1, 128), jnp.int32),
            pltpu.VMEM((1, 128), jnp.int32),
            pltpu.VMEM((1, 128), jnp.int32),
            pltpu.VMEM((BB, 16), jnp.float32),
            pltpu.VMEM((BB, 16), jnp.float32),
            pltpu.VMEM((BB, C), jnp.float32),
            pltpu.VMEM((128, 16), jnp.float32),
            pltpu.VMEM((C,), jnp.float32),
            pltpu.VMEM_SHARED((NPAD, C), jnp.float32),
            pltpu.VMEM_SHARED((NPAD, 16), jnp.float32),
            pltpu.SemaphoreType.DMA,
            pltpu.SemaphoreType.DMA,
            pltpu.SemaphoreType.DMA,
        ],
    )
    out, _ex = f(hx, asrc16, adst16, src2d, dst2d, bias_h)
    return out


def _mm1_body(x_ref, w_ref, o_ref):
    o_ref[0] = jnp.dot(x_ref[...], w_ref[...],
                       preferred_element_type=jnp.float32)


def _mm1(x, wcat, bm=512):
    nj = wcat.shape[1] // 128
    grid = (NPAD // bm, nj)
    return pl.pallas_call(
        _mm1_body,
        grid=grid,
        in_specs=[
            pl.BlockSpec((bm, x.shape[1]), lambda i, j: (i, 0)),
            pl.BlockSpec((x.shape[1], 128), lambda i, j: (0, j)),
        ],
        out_specs=pl.BlockSpec((1, bm, 128), lambda i, j: (j, i, 0)),
        out_shape=jax.ShapeDtypeStruct((nj, NPAD, 128), jnp.float32),
    )(x, wcat)


def _mm2_body(x_ref, w_ref, o_ref):
    k = pl.program_id(2)
    p = jnp.dot(x_ref[0], w_ref[...], preferred_element_type=jnp.float32)

    @pl.when(k == 0)
    def _():
        o_ref[0] = p

    @pl.when(k != 0)
    def _():
        o_ref[0] = o_ref[0] + p


def _mm2(x_heads, wcat, bm=512):
    nj = wcat.shape[1] // 128
    grid = (NPAD // bm, nj, H)
    return pl.pallas_call(
        _mm2_body,
        grid=grid,
        in_specs=[
            pl.BlockSpec((1, bm, 128), lambda i, j, k: (k, i, 0)),
            pl.BlockSpec((128, 128), lambda i, j, k: (k, j)),
        ],
        out_specs=pl.BlockSpec((1, bm, 128), lambda i, j, k: (j, i, 0)),
        out_shape=jax.ShapeDtypeStruct((nj, NPAD, 128), jnp.float32),
    )(x_heads, wcat)


def _ep_body(x_ref, b_ref, o_ref):
    o_ref[...] = jnp.mean(x_ref[...], axis=0) + b_ref[...]


def _epilogue(out_heads, b2, bm=400):
    return pl.pallas_call(
        _ep_body,
        grid=(N // bm,),
        in_specs=[
            pl.BlockSpec((H, bm, C), lambda i: (0, i, 0)),
            pl.BlockSpec((1, C), lambda i: (0, 0)),
        ],
        out_specs=pl.BlockSpec((bm, C), lambda i: (i, 0)),
        out_shape=jax.ShapeDtypeStruct((N, C), jnp.float32),
    )(out_heads, b2.reshape(1, C))


def _att_fold(W, att_src, att_dst):
    Wr = W.reshape(W.shape[0], H, C)
    wsrc = jnp.einsum("khc,hc->kh", Wr, att_src)
    wdst = jnp.einsum("khc,hc->kh", Wr, att_dst)
    pad = jnp.zeros((W.shape[0], 112), jnp.float32)
    return jnp.concatenate([W, wsrc, wdst, pad], axis=1)


def kernel(x, edge_index, W1, att_src1, att_dst1, b1, W2, att_src2,
           att_dst2, b2):
    loop = jnp.arange(N, dtype=edge_index.dtype)
    src = jnp.concatenate([edge_index[0], loop]).astype(jnp.int32)
    dst = jnp.concatenate([edge_index[1], loop]).astype(jnp.int32)
    npad_e = EP - src.shape[0]
    fill = jnp.full((npad_e,), DUMMY, jnp.int32)
    src2d = jnp.concatenate([src, fill]).reshape(EPR, 128)
    dst2d = jnp.concatenate([dst, fill]).reshape(EPR, 128)

    x_pad = jnp.zeros((NPAD, DIN), jnp.float32).at[:N].set(x)

    mm1 = _mm1(x_pad, _att_fold(W1, att_src1, att_dst1))
    hx1 = mm1[:H]
    a1 = mm1[H]
    asrc1 = jnp.tile(a1[:, 0:8], (1, 2))
    adst1 = jnp.tile(a1[:, 8:16], (1, 2))
    out1 = _sc_layer(hx1, asrc1, adst1, src2d, dst2d,
                     b1.reshape(H, C), elu=True)

    mm2 = _mm2(out1, _att_fold(W2, att_src2, att_dst2))
    hx2 = mm2[:H]
    a2 = mm2[H]
    asrc2 = jnp.tile(a2[:, 0:8], (1, 2))
    adst2 = jnp.tile(a2[:, 8:16], (1, 2))
    out2 = _sc_layer(hx2, asrc2, adst2, src2d, dst2d,
                     jnp.zeros((H, C), jnp.float32), elu=False)

    return _epilogue(out2, b2)


# speedup vs baseline: 9.2271x; 1.0477x over previous
import functools

import jax
import jax.numpy as jnp
from jax import lax
from jax.experimental import pallas as pl
from jax.experimental.pallas import tpu as pltpu
from jax.experimental.pallas import tpu_sc as plsc

N = 10000
E = 320000
H = 8
C = 128
DIN = 128

NPAD = 10240
DUMMY = N
NS = 16
NC = 2
CH = 20992
EP = NS * CH
EPR = EP // 128
BA = 256
KA = BA // 128
NBA = CH // BA
BB = 128
NSB = CH // BB
RPT = NPAD // NS


def _zero_fbuf(fbuf):
    @pl.loop(0, 128)
    def _(r):
        for k in range(8):
            fbuf[r, pl.ds(k * 16, 16)] = jnp.zeros((16,), jnp.float32)


def _sc_body(hx_hbm, asrc_hbm, adst_hbm, src_hbm, dst_hbm, bias_hbm,
             out_hbm, ex_hbm,
             ia_src, ia_dst, ga_src, ga_dst, ex_a,
             ibsA, ibdA, ibsB, ibdB, exbA, exbB, gb, dnb, bias_v,
             acc_sp, dn_sp, sem, s0, s1, s2, *, elu):
    c = lax.axis_index("c")
    s = lax.axis_index("s")
    h0 = c * 4
    estart = s * CH
    rstart = s * (CH // 128)
    r0 = s * RPT

    _zero_fbuf(gb)

    @pl.loop(0, 128)
    def _(r):
        dnb[r, :] = jnp.zeros((16,), jnp.float32)

    for rc in range(RPT // 128):
        pltpu.sync_copy(gb, acc_sp.at[pl.ds(r0 + rc * 128, 128), :])
        pltpu.sync_copy(dnb, dn_sp.at[pl.ds(r0 + rc * 128, 128), :])
    plsc.subcore_barrier()

    @pl.loop(0, NBA)
    def _(ba):
        off = estart + ba * BA
        row = rstart + ba * KA
        pltpu.sync_copy(src_hbm.at[pl.ds(row, KA), :], ia_src)
        pltpu.sync_copy(dst_hbm.at[pl.ds(row, KA), :], ia_dst)
        cps = []
        for j in range(KA):
            cps.append(pltpu.async_copy(
                asrc_hbm.at[ia_src.at[j]],
                ga_src.at[pl.ds(j * 128, 128), :], sem))
            cps.append(pltpu.async_copy(
                adst_hbm.at[ia_dst.at[j]],
                ga_dst.at[pl.ds(j * 128, 128), :], sem))
        for cp in cps:
            cp.wait()

        @pl.loop(0, BA, unroll=2)
        def _(i):
            v = ga_src[i, :] + ga_dst[i, :]
            v = jnp.where(v > 0.0, v, 0.2 * v)
            ex_a[i, :] = jnp.exp(v)

        pltpu.sync_copy(ex_a, ex_hbm.at[c, pl.ds(off, BA), :])
        for j in range(KA):
            pltpu.sync_copy(ex_a.at[pl.ds(j * 128, 128), :],
                            dn_sp.at[ia_dst.at[j]], add=True)

    plsc.subcore_barrier()

    g0 = gb.at[pl.ds(0, 64), :]
    g1 = gb.at[pl.ds(64, 64), :]

    def _gissue(hxh, ib, lo, sl, sm):
        pltpu.async_copy(hxh.at[ib.at[0, pl.ds(lo, 64)]], sl, sm)

    def _gwait(hxh, ib, lo, sl, sm):
        pltpu.make_async_copy(hxh.at[ib.at[0, pl.ds(lo, 64)]], sl, sm).wait()

    def _ldidx(row, ibs, ibd, off, exb):
        pltpu.sync_copy(src_hbm.at[pl.ds(row, 1), :], ibs)
        pltpu.sync_copy(dst_hbm.at[pl.ds(row, 1), :], ibd)
        pltpu.sync_copy(ex_hbm.at[c, pl.ds(off, BB), :], exb)

    def _lda(row, ibs, ibd, off, exb):
        pltpu.async_copy(src_hbm.at[pl.ds(row, 1), :], ibs, s2)
        pltpu.async_copy(dst_hbm.at[pl.ds(row, 1), :], ibd, s2)
        pltpu.async_copy(ex_hbm.at[c, pl.ds(off, BB), :], exb, s2)

    def _ldw(row, ibs, ibd, off, exb):
        pltpu.make_async_copy(src_hbm.at[pl.ds(row, 1), :], ibs, s2).wait()
        pltpu.make_async_copy(dst_hbm.at[pl.ds(row, 1), :], ibd, s2).wait()
        pltpu.make_async_copy(ex_hbm.at[c, pl.ds(off, BB), :], exb,
                              s2).wait()

    @pl.loop(0, 4)
    def _(hl):
        h = h0 + hl
        hxh = hx_hbm.at[h]
        hvec = jnp.full((16,), h, jnp.int32)

        def _scale(base, exb):
            @pl.loop(0, 64, unroll=2)
            def _(e):
                w = plsc.load_gather(
                    exb, [jnp.full((16,), e, jnp.int32) + base, hvec])
                for k in range(8):
                    gb[base + e, pl.ds(k * 16, 16)] = (
                        gb[base + e, pl.ds(k * 16, 16)] * w)

        def _proc(ibs, ibd, exb, hxh=hxh):
            _gwait(hxh, ibs, 0, g0, s0)
            _scale(0, exb)
            _gwait(hxh, ibs, 64, g1, s1)
            _scale(64, exb)
            pltpu.sync_copy(gb, acc_sp.at[ibd.at[0]], add=True)

        def _fire(ibs, hxh=hxh):
            _gissue(hxh, ibs, 0, g0, s0)
            _gissue(hxh, ibs, 64, g1, s1)

        _ldidx(rstart, ibsA, ibdA, estart, exbA)
        _fire(ibsA)

        @pl.loop(0, NSB // 2 - 1)
        def _(j):
            row = rstart + 2 * j
            off = estart + j * (2 * BB)
            _lda(row + 1, ibsB, ibdB, off + BB, exbB)
            _proc(ibsA, ibdA, exbA)
            _ldw(row + 1, ibsB, ibdB, off + BB, exbB)
            _fire(ibsB)
            _lda(row + 2, ibsA, ibdA, off + 2 * BB, exbA)
            _proc(ibsB, ibdB, exbB)
            _ldw(row + 2, ibsA, ibdA, off + 2 * BB, exbA)
            _fire(ibsA)

        _lda(rstart + NSB - 1, ibsB, ibdB, estart + (NSB - 1) * BB, exbB)
        _proc(ibsA, ibdA, exbA)
        _ldw(rstart + NSB - 1, ibsB, ibdB, estart + (NSB - 1) * BB, exbB)
        _fire(ibsB)
        _proc(ibsB, ibdB, exbB)

        plsc.subcore_barrier()

        pltpu.sync_copy(bias_hbm.at[h], bias_v)
        hvec = jnp.full((16,), h, jnp.int32)
        for rc in range(RPT // 128):
            rr = r0 + rc * 128
            pltpu.sync_copy(acc_sp.at[pl.ds(rr, 128), :], gb)
            pltpu.sync_copy(dn_sp.at[pl.ds(rr, 128), :], dnb)

            @pl.loop(0, 128)
            def _(r):
                d = plsc.load_gather(
                    dnb, [jnp.full((16,), r, jnp.int32), hvec])
                d = 1.0 / (d + 1e-16)
                for k in range(8):
                    v = gb[r, pl.ds(k * 16, 16)] * d
                    v = v + bias_v[pl.ds(k * 16, 16)]
                    if elu:
                        v = jnp.where(
                            v > 0.0, v,
                            jnp.exp(jnp.minimum(v, 0.0)) - 1.0)
                    gb[r, pl.ds(k * 16, 16)] = v

            pltpu.sync_copy(gb, out_hbm.at[h, pl.ds(rr, 128), :])

        _zero_fbuf(gb)
        for rc in range(RPT // 128):
            pltpu.sync_copy(gb, acc_sp.at[pl.ds(r0 + rc * 128, 128), :])
        plsc.subcore_barrier()


def _sc_layer(hx, asrc16, adst16, src2d, dst2d, bias_h, elu):
    mesh = plsc.VectorSubcoreMesh(core_axis_name="c", subcore_axis_name="s",
                                  num_cores=NC, num_subcores=NS)
    f = pl.kernel(
        functools.partial(_sc_body, elu=elu),
        out_type=(
            jax.ShapeDtypeStruct((H, NPAD, C), jnp.float32),
            jax.ShapeDtypeStruct((NC, EP, 16), jnp.float32),
        ),
        mesh=mesh,
        compiler_params=pltpu.CompilerParams(needs_layout_passes=False,
                                             use_tc_tiling_on_sc=False),
        scratch_types=[
            pltpu.VMEM((KA, 128), jnp.int32),
            pltpu.VMEM((KA, 128), jnp.int32),
            pltpu.VMEM((BA, 16), jnp.float32),
            pltpu.VMEM((BA, 16), jnp.float32),
            pltpu.VMEM((BA, 16), jnp.float32),
            pltpu.VMEM((1, 128), jnp.int32),
            pltpu.VMEM((1, 128), jnp.int32),
            pltpu.VMEM((1, 128), jnp.int32),
            pltpu.VMEM((1, 128), jnp.int32),
            pltpu.VMEM((BB, 16), jnp.float32),
            pltpu.VMEM((BB, 16), jnp.float32),
            pltpu.VMEM((BB, C), jnp.float32),
            pltpu.VMEM((128, 16), jnp.float32),
            pltpu.VMEM((C,), jnp.float32),
            pltpu.VMEM_SHARED((NPAD, C), jnp.float32),
            pltpu.VMEM_SHARED((NPAD, 16), jnp.float32),
            pltpu.SemaphoreType.DMA,
            pltpu.SemaphoreType.DMA,
            pltpu.SemaphoreType.DMA,
            pltpu.SemaphoreType.DMA,
        ],
    )
    out, _ex = f(hx, asrc16, adst16, src2d, dst2d, bias_h)
    return out


def _mm1_body(x_ref, w_ref, o_ref):
    o_ref[0] = jnp.dot(x_ref[...], w_ref[...],
                       preferred_element_type=jnp.float32)


def _mm1(x, wcat, bm=512):
    nj = wcat.shape[1] // 128
    grid = (NPAD // bm, nj)
    return pl.pallas_call(
        _mm1_body,
        grid=grid,
        in_specs=[
            pl.BlockSpec((bm, x.shape[1]), lambda i, j: (i, 0)),
            pl.BlockSpec((x.shape[1], 128), lambda i, j: (0, j)),
        ],
        out_specs=pl.BlockSpec((1, bm, 128), lambda i, j: (j, i, 0)),
        out_shape=jax.ShapeDtypeStruct((nj, NPAD, 128), jnp.float32),
    )(x, wcat)


def _mm2_body(x_ref, w_ref, o_ref):
    k = pl.program_id(2)
    p = jnp.dot(x_ref[0], w_ref[...], preferred_element_type=jnp.float32)

    @pl.when(k == 0)
    def _():
        o_ref[0] = p

    @pl.when(k != 0)
    def _():
        o_ref[0] = o_ref[0] + p


def _mm2(x_heads, wcat, bm=512):
    nj = wcat.shape[1] // 128
    grid = (NPAD // bm, nj, H)
    return pl.pallas_call(
        _mm2_body,
        grid=grid,
        in_specs=[
            pl.BlockSpec((1, bm, 128), lambda i, j, k: (k, i, 0)),
            pl.BlockSpec((128, 128), lambda i, j, k: (k, j)),
        ],
        out_specs=pl.BlockSpec((1, bm, 128), lambda i, j, k: (j, i, 0)),
        out_shape=jax.ShapeDtypeStruct((nj, NPAD, 128), jnp.float32),
    )(x_heads, wcat)


def _ep_body(x_ref, b_ref, o_ref):
    o_ref[...] = jnp.mean(x_ref[...], axis=0) + b_ref[...]


def _epilogue(out_heads, b2, bm=400):
    return pl.pallas_call(
        _ep_body,
        grid=(N // bm,),
        in_specs=[
            pl.BlockSpec((H, bm, C), lambda i: (0, i, 0)),
            pl.BlockSpec((1, C), lambda i: (0, 0)),
        ],
        out_specs=pl.BlockSpec((bm, C), lambda i: (i, 0)),
        out_shape=jax.ShapeDtypeStruct((N, C), jnp.float32),
    )(out_heads, b2.reshape(1, C))


def _att_fold(W, att_src, att_dst):
    Wr = W.reshape(W.shape[0], H, C)
    wsrc = jnp.einsum("khc,hc->kh", Wr, att_src)
    wdst = jnp.einsum("khc,hc->kh", Wr, att_dst)
    pad = jnp.zeros((W.shape[0], 112), jnp.float32)
    return jnp.concatenate([W, wsrc, wdst, pad], axis=1)


def kernel(x, edge_index, W1, att_src1, att_dst1, b1, W2, att_src2,
           att_dst2, b2):
    loop = jnp.arange(N, dtype=edge_index.dtype)
    src = jnp.concatenate([edge_index[0], loop]).astype(jnp.int32)
    dst = jnp.concatenate([edge_index[1], loop]).astype(jnp.int32)
    npad_e = EP - src.shape[0]
    fill = jnp.full((npad_e,), DUMMY, jnp.int32)
    src2d = jnp.concatenate([src, fill]).reshape(EPR, 128)
    dst2d = jnp.concatenate([dst, fill]).reshape(EPR, 128)

    x_pad = jnp.zeros((NPAD, DIN), jnp.float32).at[:N].set(x)

    mm1 = _mm1(x_pad, _att_fold(W1, att_src1, att_dst1))
    hx1 = mm1[:H]
    a1 = mm1[H]
    asrc1 = jnp.tile(a1[:, 0:8], (1, 2))
    adst1 = jnp.tile(a1[:, 8:16], (1, 2))
    out1 = _sc_layer(hx1, asrc1, adst1, src2d, dst2d,
                     b1.reshape(H, C), elu=True)

    mm2 = _mm2(out1, _att_fold(W2, att_src2, att_dst2))
    hx2 = mm2[:H]
    a2 = mm2[H]
    asrc2 = jnp.tile(a2[:, 0:8], (1, 2))
    adst2 = jnp.tile(a2[:, 8:16], (1, 2))
    out2 = _sc_layer(hx2, asrc2, adst2, src2d, dst2d,
                     jnp.zeros((H, C), jnp.float32), elu=False)

    return _epilogue(out2, b2)


# speedup vs baseline: 11.4479x; 1.2407x over previous
import functools

import jax
import jax.numpy as jnp
from jax import lax
from jax.experimental import pallas as pl
from jax.experimental.pallas import tpu as pltpu
from jax.experimental.pallas import tpu_sc as plsc

N = 10000
E = 320000
H = 8
C = 128
DIN = 128

NPAD = 10240
DUMMY = N
NS = 16
NC = 2
CH = 20992
EP = NS * CH
EPR = EP // 128
BB = 128
NSB = CH // BB
RPT = NPAD // NS


def _zero_fbuf(fbuf):
    @pl.loop(0, 128)
    def _(r):
        for k in range(8):
            fbuf[r, pl.ds(k * 16, 16)] = jnp.zeros((16,), jnp.float32)


def _sc_body(hx_hbm, asrc_hbm, adst_hbm, src_hbm, dst_hbm, bias_hbm,
             out_hbm, ex_hbm,
             ia_src, ia_dst,
             ibs0, ibd0, ibs1, ibd1, ibs2, ibd2, ibs3, ibd3,
             exb0, exb1, gb, gb2, bias_v,
             acc_sp, dn_sp, sem, s0, s1, s2, s3, s4, s5, *, elu):
    c = lax.axis_index("c")
    s = lax.axis_index("s")
    h0 = c * 4
    estart = s * CH
    rstart = s * NSB
    r0 = s * RPT

    ibs = [ibs0, ibs1, ibs2, ibs3]
    ibd = [ibd0, ibd1, ibd2, ibd3]
    exb = [exb0, exb1]
    gbs = [gb, gb2]
    gsem = [(s0, s1), (s2, s3)]
    psem = [s4, s5]

    _zero_fbuf(gb)

    @pl.loop(0, 128)
    def _(r):
        exb0[r, :] = jnp.zeros((16,), jnp.float32)

    for rc in range(RPT // 128):
        pltpu.sync_copy(gb, acc_sp.at[pl.ds(r0 + rc * 128, 128), :])
        pltpu.sync_copy(exb0, dn_sp.at[pl.ds(r0 + rc * 128, 128), :])
    plsc.subcore_barrier()

    @pl.loop(0, NSB)
    def _(ba):
        off = estart + ba * BB
        row = rstart + ba
        pltpu.sync_copy(src_hbm.at[pl.ds(row, 1), :], ia_src)
        pltpu.sync_copy(dst_hbm.at[pl.ds(row, 1), :], ia_dst)
        cp1 = pltpu.async_copy(asrc_hbm.at[ia_src.at[0]], exb0, sem)
        cp2 = pltpu.async_copy(adst_hbm.at[ia_dst.at[0]], exb1, sem)
        cp1.wait()
        cp2.wait()

        @pl.loop(0, BB, unroll=2)
        def _(i):
            v = exb0[i, :] + exb1[i, :]
            v = jnp.where(v > 0.0, v, 0.2 * v)
            exb0[i, :] = jnp.exp(v)

        pltpu.sync_copy(exb0, ex_hbm.at[c, pl.ds(off, BB), :])
        pltpu.sync_copy(exb0, dn_sp.at[ia_dst.at[0]], add=True)

    plsc.subcore_barrier()

    def _fire(hxh, sb_ibs, g, sems):
        pltpu.async_copy(hxh.at[sb_ibs.at[0, pl.ds(0, 64)]],
                         g.at[pl.ds(0, 64), :], sems[0])
        pltpu.async_copy(hxh.at[sb_ibs.at[0, pl.ds(64, 64)]],
                         g.at[pl.ds(64, 64), :], sems[1])

    def _pf(row, si, off, eb, ps):
        pltpu.async_copy(src_hbm.at[pl.ds(row, 1), :], ibs[si], ps)
        pltpu.async_copy(dst_hbm.at[pl.ds(row, 1), :], ibd[si], ps)
        pltpu.async_copy(ex_hbm.at[c, pl.ds(off, BB), :], eb, ps)

    def _pfw(row, si, off, eb, ps):
        pltpu.make_async_copy(src_hbm.at[pl.ds(row, 1), :], ibs[si],
                              ps).wait()
        pltpu.make_async_copy(dst_hbm.at[pl.ds(row, 1), :], ibd[si],
                              ps).wait()
        pltpu.make_async_copy(ex_hbm.at[c, pl.ds(off, BB), :], eb,
                              ps).wait()

    def _pfe(off, eb, ps):
        pltpu.async_copy(ex_hbm.at[c, pl.ds(off, BB), :], eb, ps)

    def _pfew(off, eb, ps):
        pltpu.make_async_copy(ex_hbm.at[c, pl.ds(off, BB), :], eb,
                              ps).wait()

    @pl.loop(0, 4)
    def _(hl):
        h = h0 + hl
        hxh = hx_hbm.at[h]
        hvec = jnp.full((16,), h, jnp.int32)

        def _scale(g, base, eb):
            @pl.loop(0, 64, unroll=2)
            def _(e):
                w = plsc.load_gather(
                    eb, [jnp.full((16,), e, jnp.int32) + base, hvec])
                for k in range(8):
                    g[base + e, pl.ds(k * 16, 16)] = (
                        g[base + e, pl.ds(k * 16, 16)] * w)

        def _proc(si, g, eb, sems, hxh=hxh):
            pltpu.make_async_copy(
                hxh.at[ibs[si].at[0, pl.ds(0, 64)]],
                g.at[pl.ds(0, 64), :], sems[0]).wait()
            _scale(g, 0, eb)
            pltpu.make_async_copy(
                hxh.at[ibs[si].at[0, pl.ds(64, 64)]],
                g.at[pl.ds(64, 64), :], sems[1]).wait()
            _scale(g, 64, eb)
            pltpu.sync_copy(g, acc_sp.at[ibd[si].at[0]], add=True)

        pltpu.sync_copy(src_hbm.at[pl.ds(rstart, 1), :], ibs0)
        pltpu.sync_copy(dst_hbm.at[pl.ds(rstart, 1), :], ibd0)
        pltpu.sync_copy(src_hbm.at[pl.ds(rstart + 1, 1), :], ibs1)
        pltpu.sync_copy(dst_hbm.at[pl.ds(rstart + 1, 1), :], ibd1)
        _pf(rstart + 2, 2, estart, exb0, s4)
        _pf(rstart + 3, 3, estart + BB, exb1, s5)
        _fire(hxh, ibs0, gb, gsem[0])
        _fire(hxh, ibs1, gb2, gsem[1])

        @pl.loop(0, NSB // 4 - 1)
        def _(kk):
            m0 = 4 * kk
            for i in range(4):
                m = m0 + i
                row = rstart + m
                off = estart + m * BB
                par = i % 2
                _pfw(row + 2, (i + 2) % 4, off, exb[par], psem[par])
                _proc(i, gbs[par], exb[par], gsem[par])
                _fire(hxh, ibs[(i + 2) % 4], gbs[par], gsem[par])
                _pf(row + 4, i, off + 2 * BB, exb[par], psem[par])

        for i in range(4):
            m = NSB - 4 + i
            row = rstart + m
            off = estart + m * BB
            par = i % 2
            if i < 2:
                _pfw(row + 2, (i + 2) % 4, off, exb[par], psem[par])
            else:
                _pfew(off, exb[par], psem[par])
            _proc(i, gbs[par], exb[par], gsem[par])
            if i < 2:
                _fire(hxh, ibs[(i + 2) % 4], gbs[par], gsem[par])
                _pfe(off + 2 * BB, exb[par], psem[par])

        plsc.subcore_barrier()

        pltpu.sync_copy(bias_hbm.at[h], bias_v)
        for rc in range(RPT // 128):
            rr = r0 + rc * 128
            pltpu.sync_copy(acc_sp.at[pl.ds(rr, 128), :], gb)
            pltpu.sync_copy(dn_sp.at[pl.ds(rr, 128), :], exb0)

            @pl.loop(0, 128)
            def _(r):
                d = plsc.load_gather(
                    exb0, [jnp.full((16,), r, jnp.int32), hvec])
                d = 1.0 / (d + 1e-16)
                for k in range(8):
                    v = gb[r, pl.ds(k * 16, 16)] * d
                    v = v + bias_v[pl.ds(k * 16, 16)]
                    if elu:
                        v = jnp.where(
                            v > 0.0, v,
                            jnp.exp(jnp.minimum(v, 0.0)) - 1.0)
                    gb[r, pl.ds(k * 16, 16)] = v

            pltpu.sync_copy(gb, out_hbm.at[h, pl.ds(rr, 128), :])

        _zero_fbuf(gb)
        for rc in range(RPT // 128):
            pltpu.sync_copy(gb, acc_sp.at[pl.ds(r0 + rc * 128, 128), :])
        plsc.subcore_barrier()


def _sc_layer(hx, asrc16, adst16, src2d, dst2d, bias_h, elu):
    mesh = plsc.VectorSubcoreMesh(core_axis_name="c", subcore_axis_name="s",
                                  num_cores=NC, num_subcores=NS)
    f = pl.kernel(
        functools.partial(_sc_body, elu=elu),
        out_type=(
            jax.ShapeDtypeStruct((H, NPAD, C), jnp.float32),
            jax.ShapeDtypeStruct((NC, EP, 16), jnp.float32),
        ),
        mesh=mesh,
        compiler_params=pltpu.CompilerParams(needs_layout_passes=False,
                                             use_tc_tiling_on_sc=False),
        scratch_types=[
            pltpu.VMEM((1, 128), jnp.int32),
            pltpu.VMEM((1, 128), jnp.int32),
            pltpu.VMEM((1, 128), jnp.int32),
            pltpu.VMEM((1, 128), jnp.int32),
            pltpu.VMEM((1, 128), jnp.int32),
            pltpu.VMEM((1, 128), jnp.int32),
            pltpu.VMEM((1, 128), jnp.int32),
            pltpu.VMEM((1, 128), jnp.int32),
            pltpu.VMEM((1, 128), jnp.int32),
            pltpu.VMEM((1, 128), jnp.int32),
            pltpu.VMEM((BB, 16), jnp.float32),
            pltpu.VMEM((BB, 16), jnp.float32),
            pltpu.VMEM((BB, C), jnp.float32),
            pltpu.VMEM((BB, C), jnp.float32),
            pltpu.VMEM((C,), jnp.float32),
            pltpu.VMEM_SHARED((NPAD, C), jnp.float32),
            pltpu.VMEM_SHARED((NPAD, 16), jnp.float32),
            pltpu.SemaphoreType.DMA,
            pltpu.SemaphoreType.DMA,
            pltpu.SemaphoreType.DMA,
            pltpu.SemaphoreType.DMA,
            pltpu.SemaphoreType.DMA,
            pltpu.SemaphoreType.DMA,
            pltpu.SemaphoreType.DMA,
        ],
    )
    out, _ex = f(hx, asrc16, adst16, src2d, dst2d, bias_h)
    return out


def _mm1_body(x_ref, w_ref, o_ref):
    o_ref[0] = jnp.dot(x_ref[...], w_ref[...],
                       preferred_element_type=jnp.float32)


def _mm1(x, wcat, bm=512):
    nj = wcat.shape[1] // 128
    grid = (NPAD // bm, nj)
    return pl.pallas_call(
        _mm1_body,
        grid=grid,
        in_specs=[
            pl.BlockSpec((bm, x.shape[1]), lambda i, j: (i, 0)),
            pl.BlockSpec((x.shape[1], 128), lambda i, j: (0, j)),
        ],
        out_specs=pl.BlockSpec((1, bm, 128), lambda i, j: (j, i, 0)),
        out_shape=jax.ShapeDtypeStruct((nj, NPAD, 128), jnp.float32),
    )(x, wcat)


def _mm2_body(x_ref, w_ref, o_ref):
    k = pl.program_id(2)
    p = jnp.dot(x_ref[0], w_ref[...], preferred_element_type=jnp.float32)

    @pl.when(k == 0)
    def _():
        o_ref[0] = p

    @pl.when(k != 0)
    def _():
        o_ref[0] = o_ref[0] + p


def _mm2(x_heads, wcat, bm=512):
    nj = wcat.shape[1] // 128
    grid = (NPAD // bm, nj, H)
    return pl.pallas_call(
        _mm2_body,
        grid=grid,
        in_specs=[
            pl.BlockSpec((1, bm, 128), lambda i, j, k: (k, i, 0)),
            pl.BlockSpec((128, 128), lambda i, j, k: (k, j)),
        ],
        out_specs=pl.BlockSpec((1, bm, 128), lambda i, j, k: (j, i, 0)),
        out_shape=jax.ShapeDtypeStruct((nj, NPAD, 128), jnp.float32),
    )(x_heads, wcat)


def _ep_body(x_ref, b_ref, o_ref):
    o_ref[...] = jnp.mean(x_ref[...], axis=0) + b_ref[...]


def _epilogue(out_heads, b2, bm=400):
    return pl.pallas_call(
        _ep_body,
        grid=(N // bm,),
        in_specs=[
            pl.BlockSpec((H, bm, C), lambda i: (0, i, 0)),
            pl.BlockSpec((1, C), lambda i: (0, 0)),
        ],
        out_specs=pl.BlockSpec((bm, C), lambda i: (i, 0)),
        out_shape=jax.ShapeDtypeStruct((N, C), jnp.float32),
    )(out_heads, b2.reshape(1, C))


def _att_fold(W, att_src, att_dst):
    Wr = W.reshape(W.shape[0], H, C)
    wsrc = jnp.einsum("khc,hc->kh", Wr, att_src)
    wdst = jnp.einsum("khc,hc->kh", Wr, att_dst)
    pad = jnp.zeros((W.shape[0], 112), jnp.float32)
    return jnp.concatenate([W, wsrc, wdst, pad], axis=1)


def kernel(x, edge_index, W1, att_src1, att_dst1, b1, W2, att_src2,
           att_dst2, b2):
    loop = jnp.arange(N, dtype=edge_index.dtype)
    src = jnp.concatenate([edge_index[0], loop]).astype(jnp.int32)
    dst = jnp.concatenate([edge_index[1], loop]).astype(jnp.int32)
    npad_e = EP - src.shape[0]
    fill = jnp.full((npad_e,), DUMMY, jnp.int32)
    src2d = jnp.concatenate([src, fill]).reshape(EPR, 128)
    dst2d = jnp.concatenate([dst, fill]).reshape(EPR, 128)

    x_pad = jnp.zeros((NPAD, DIN), jnp.float32).at[:N].set(x)

    mm1 = _mm1(x_pad, _att_fold(W1, att_src1, att_dst1))
    hx1 = mm1[:H]
    a1 = mm1[H]
    asrc1 = jnp.tile(a1[:, 0:8], (1, 2))
    adst1 = jnp.tile(a1[:, 8:16], (1, 2))
    out1 = _sc_layer(hx1, asrc1, adst1, src2d, dst2d,
                     b1.reshape(H, C), elu=True)

    mm2 = _mm2(out1, _att_fold(W2, att_src2, att_dst2))
    hx2 = mm2[:H]
    a2 = mm2[H]
    asrc2 = jnp.tile(a2[:, 0:8], (1, 2))
    adst2 = jnp.tile(a2[:, 8:16], (1, 2))
    out2 = _sc_layer(hx2, asrc2, adst2, src2d, dst2d,
                     jnp.zeros((H, C), jnp.float32), elu=False)

    return _epilogue(out2, b2)
